# baseline (speedup 1.0000x reference)
import functools

import jax
import jax.numpy as jnp
from jax import lax
from jax.experimental import pallas as pl
from jax.experimental.pallas import tpu as pltpu
from jax.experimental.pallas import tpu_sc as plsc

B = 4
H = 256
N = 2048
M = 2046
MP = 2048
L = 16
NC = 2
NS = 16
NW = NC * NS
WPB = NW // B

XA = 256
YA = 256
XVPW = XA // WPB // L
YVPW = YA // WPB // L
SCALE_X = 1.0 / (B * N)
SCALE_Y = 1.0 / (B * M)


def _tail_body(rows_ref, ldr_ref, out_ref, *, k_lo, k_hi, j_lo, j_hi):
    r = rows_ref[0, 0]
    c = r[5:6, :] + r[6:7, :] + r[7:8, :]
    c1 = pltpu.roll(c, N - 1, axis=1)
    c2 = pltpu.roll(c, N - 2, axis=1)
    mid = (c + c1 + c2) * (1.0 / 9.0)
    lane = lax.broadcasted_iota(jnp.int32, (1, MP), 1)
    mid = jnp.where(lane < M, mid, jnp.inf)

    ldT = jnp.transpose(ldr_ref[0], (1, 0))
    inf128 = jnp.full((128, 128), jnp.inf, jnp.float32)
    laneid = lax.broadcasted_iota(jnp.int32, (1, 128), 1)

    sx = jnp.float32(0.0)
    for k in range(k_lo, k_hi):
        xc = ldT[:, k:k + 1]
        acc = inf128
        for j in range(MP // 128):
            yb = mid[:, j * 128:(j + 1) * 128]
            acc = jnp.minimum(acc, (xc - yb) ** 2)
        sx = sx + jnp.sum(jnp.min(acc, axis=1))

    sy = jnp.float32(0.0)
    for j in range(j_lo, j_hi):
        yb = mid[:, j * 128:(j + 1) * 128]
        acc = inf128
        for k in range(N // 128):
            xc = ldT[:, k:k + 1]
            acc = jnp.minimum(acc, (xc - yb) ** 2)
        v = jnp.min(acc, axis=0, keepdims=True)
        v = jnp.where(laneid + j * 128 < M, v, 0.0)
        sy = sy + jnp.sum(v)

    val = sx * SCALE_X + sy * SCALE_Y
    subl = lax.broadcasted_iota(jnp.int32, (8, 128), 0)
    lane8 = lax.broadcasted_iota(jnp.int32, (8, 128), 1)
    out_ref[...] = jnp.where((subl == 0) & (lane8 == 0), val, 0.0)


SPLIT = 8


def _tail(output, ldr, k_lo, k_hi, j_lo, j_hi):
    body = functools.partial(
        _tail_body, k_lo=k_lo, k_hi=k_hi, j_lo=j_lo, j_hi=j_hi
    )
    return pl.pallas_call(
        body,
        grid=(B,),
        in_specs=[
            pl.BlockSpec((1, 1, 8, N), lambda b: (b, 0, (H // 8) - 1, 0)),
            pl.BlockSpec((1, N // 128, 128), lambda b: (b, 0, 0)),
        ],
        out_specs=pl.BlockSpec((8, 128), lambda b: (b, 0)),
        out_shape=jax.ShapeDtypeStruct((B * 8, 128), jnp.float32),
    )(output, ldr)


def _iota16():
    return lax.iota(jnp.int32, L)


_GDN = lax.GatherDimensionNumbers(
    offset_dims=(), collapsed_slice_dims=(0,), start_index_map=(0,)
)


def _bcast_lane(v, l):
    idx = jnp.full((L, 1), l, jnp.int32)
    return lax.gather(v, idx, _GDN, (1,),
                      mode=lax.GatherScatterMode.PROMISE_IN_BOUNDS)


def _chamfer_body(out4_hbm, lidar_hbm, out_hbm, rows_v, c_v, midp_v, ld_v, out_v):
    wid = lax.axis_index("c") * NS + lax.axis_index("s")
    b = wid // WPB
    ci = wid % WPB

    pltpu.sync_copy(out4_hbm.at[b, 0, pl.ds(H - 3, 3)], rows_v)
    pltpu.sync_copy(lidar_hbm.at[b, 0], ld_v)

    iota = _iota16()
    inf_v = jnp.full((L,), jnp.inf, jnp.float32)

    def mid_step(i, _):
        idx = iota + i * L
        c = (plsc.load_gather(rows_v, [jnp.full((L,), 0, jnp.int32), idx])
             + plsc.load_gather(rows_v, [jnp.full((L,), 1, jnp.int32), idx])
             + plsc.load_gather(rows_v, [jnp.full((L,), 2, jnp.int32), idx]))
        plsc.store_scatter(c_v, [idx], c)
        return 0

    lax.fori_loop(0, MP // L, mid_step, 0)

    def mid_step2(i, _):
        idx = iota + i * L
        i1 = jnp.minimum(idx + 1, N - 1)
        i2 = jnp.minimum(idx + 2, N - 1)
        m = (plsc.load_gather(c_v, [idx])
             + plsc.load_gather(c_v, [i1])
             + plsc.load_gather(c_v, [i2])) * (1.0 / 9.0)
        m = jnp.where(idx < M, m, jnp.inf)
        plsc.store_scatter(midp_v, [idx], m)
        return 0

    lax.fori_loop(0, MP // L, mid_step2, 0)

    xbase = ci * (XA // WPB)
    xs = [plsc.load_gather(ld_v, [iota + (xbase + i * L)]) for i in range(XVPW)]

    def body_a(jc, accs):
        ych = plsc.load_gather(midp_v, [iota + jc * L])
        acc = list(accs)
        for l in range(L):
            yb = _bcast_lane(ych, l)
            for i in range(XVPW):
                d = xs[i] - yb
                acc[i] = jnp.minimum(acc[i], d * d)
        return tuple(acc)

    accs_a = lax.fori_loop(0, MP // L, body_a, (inf_v,) * XVPW)
    s_a = accs_a[0]
    for i in range(1, XVPW):
        s_a = s_a + accs_a[i]

    ybase = ci * (YA // WPB)
    ys = [plsc.load_gather(midp_v, [iota + (ybase + i * L)]) for i in range(YVPW)]

    def body_b(jc, accs):
        xch = plsc.load_gather(ld_v, [iota + jc * L])
        acc = list(accs)
        for l in range(L):
            xb = _bcast_lane(xch, l)
            for i in range(YVPW):
                d = ys[i] - xb
                acc[i] = jnp.minimum(acc[i], d * d)
        return tuple(acc)

    accs_b = lax.fori_loop(0, N // L, body_b, (inf_v,) * YVPW)
    s_b = accs_b[0]
    for i in range(1, YVPW):
        s_b = s_b + accs_b[i]

    out_v[...] = s_a * SCALE_X + s_b * SCALE_Y
    pltpu.sync_copy(out_v, out_hbm.at[wid])


def _chamfer_head(output, lidar):
    mesh = plsc.VectorSubcoreMesh(
        core_axis_name="c", subcore_axis_name="s", num_cores=NC, num_subcores=NS
    )
    f = pl.kernel(
        _chamfer_body,
        out_type=jax.ShapeDtypeStruct((NW, L), jnp.float32),
        mesh=mesh,
        compiler_params=pltpu.CompilerParams(needs_layout_passes=False),
        scratch_types=[
            pltpu.VMEM((3, N), jnp.float32),
            pltpu.VMEM((N,), jnp.float32),
            pltpu.VMEM((MP,), jnp.float32),
            pltpu.VMEM((N,), jnp.float32),
            pltpu.VMEM((L,), jnp.float32),
        ],
    )
    return f(output, lidar)


@jax.jit
def kernel(output, lidar):
    ldr = lidar.reshape(B, N // 128, 128)
    tail_a = _tail(output, ldr, XA // 128, SPLIT, YA // 128, SPLIT)
    ta = jnp.sum(tail_a)
    lidar_dep = jnp.minimum(lidar, jnp.inf + ta)
    head = _chamfer_head(output, lidar_dep)
    ldr_b = lidar_dep.reshape(B, N // 128, 128)
    tail_b = _tail(output, ldr_b, SPLIT, N // 128, SPLIT, MP // 128)
    return jnp.sum(head) + ta + jnp.sum(tail_b)


# speedup vs baseline: 1.2661x; 1.2661x over previous
import functools

import jax
import jax.numpy as jnp
from jax import lax
from jax.experimental import pallas as pl
from jax.experimental.pallas import tpu as pltpu
from jax.experimental.pallas import tpu_sc as plsc

B = 4
H = 256
N = 2048
M = 2046
MP = 2048
L = 16
NC = 2
NS = 16
NW = NC * NS
WPB = NW // B

XA = 256
YA = 256
XVPW = XA // WPB // L
YVPW = YA // WPB // L
SCALE_X = 1.0 / (B * N)
SCALE_Y = 1.0 / (B * M)


def _tail_body(rows_ref, ldr_ref, out_ref, *, k_lo, k_hi, j_lo, j_hi):
    r = rows_ref[0, 0]
    c = r[5:6, :] + r[6:7, :] + r[7:8, :]
    c1 = pltpu.roll(c, N - 1, axis=1)
    c2 = pltpu.roll(c, N - 2, axis=1)
    mid = (c + c1 + c2) * (1.0 / 9.0)
    lane = lax.broadcasted_iota(jnp.int32, (1, MP), 1)
    mid = jnp.where(lane < M, mid, jnp.inf)

    ldT = jnp.transpose(ldr_ref[0], (1, 0))
    inf128 = jnp.full((128, 128), jnp.inf, jnp.float32)
    laneid = lax.broadcasted_iota(jnp.int32, (1, 128), 1)

    sx = jnp.float32(0.0)
    for k in range(k_lo, k_hi, 2):
        xc0 = ldT[:, k:k + 1]
        xc1 = ldT[:, k + 1:k + 2]
        acc0 = inf128
        acc1 = inf128
        for j in range(MP // 128):
            yb = mid[:, j * 128:(j + 1) * 128]
            acc0 = jnp.minimum(acc0, (xc0 - yb) ** 2)
            acc1 = jnp.minimum(acc1, (xc1 - yb) ** 2)
        sx = sx + jnp.sum(jnp.min(acc0, axis=1)) + jnp.sum(jnp.min(acc1, axis=1))

    sy = jnp.float32(0.0)
    for j in range(j_lo, j_hi, 2):
        yb0 = mid[:, j * 128:(j + 1) * 128]
        yb1 = mid[:, (j + 1) * 128:(j + 2) * 128]
        acc0 = inf128
        acc1 = inf128
        for k in range(N // 128):
            xc = ldT[:, k:k + 1]
            acc0 = jnp.minimum(acc0, (xc - yb0) ** 2)
            acc1 = jnp.minimum(acc1, (xc - yb1) ** 2)
        v0 = jnp.min(acc0, axis=0, keepdims=True)
        v1 = jnp.min(acc1, axis=0, keepdims=True)
        v0 = jnp.where(laneid + j * 128 < M, v0, 0.0)
        v1 = jnp.where(laneid + (j + 1) * 128 < M, v1, 0.0)
        sy = sy + jnp.sum(v0) + jnp.sum(v1)

    val = sx * SCALE_X + sy * SCALE_Y
    subl = lax.broadcasted_iota(jnp.int32, (8, 128), 0)
    lane8 = lax.broadcasted_iota(jnp.int32, (8, 128), 1)
    out_ref[...] = jnp.where((subl == 0) & (lane8 == 0), val, 0.0)


SPLIT = 8


def _tail(output, ldr, k_lo, k_hi, j_lo, j_hi):
    body = functools.partial(
        _tail_body, k_lo=k_lo, k_hi=k_hi, j_lo=j_lo, j_hi=j_hi
    )
    return pl.pallas_call(
        body,
        grid=(B,),
        in_specs=[
            pl.BlockSpec((1, 1, 8, N), lambda b: (b, 0, (H // 8) - 1, 0)),
            pl.BlockSpec((1, N // 128, 128), lambda b: (b, 0, 0)),
        ],
        out_specs=pl.BlockSpec((8, 128), lambda b: (b, 0)),
        out_shape=jax.ShapeDtypeStruct((B * 8, 128), jnp.float32),
    )(output, ldr)


def _iota16():
    return lax.iota(jnp.int32, L)


_GDN = lax.GatherDimensionNumbers(
    offset_dims=(), collapsed_slice_dims=(0,), start_index_map=(0,)
)


def _bcast_lane(v, l):
    idx = jnp.full((L, 1), l, jnp.int32)
    return lax.gather(v, idx, _GDN, (1,),
                      mode=lax.GatherScatterMode.PROMISE_IN_BOUNDS)


def _chamfer_body(out4_hbm, lidar_hbm, out_hbm, rows_v, c_v, midp_v, ld_v, out_v):
    wid = lax.axis_index("c") * NS + lax.axis_index("s")
    b = wid // WPB
    ci = wid % WPB

    pltpu.sync_copy(out4_hbm.at[b, 0, pl.ds(H - 3, 3)], rows_v)
    pltpu.sync_copy(lidar_hbm.at[b, 0], ld_v)

    iota = _iota16()
    inf_v = jnp.full((L,), jnp.inf, jnp.float32)

    def mid_step(i, _):
        idx = iota + i * L
        c = (plsc.load_gather(rows_v, [jnp.full((L,), 0, jnp.int32), idx])
             + plsc.load_gather(rows_v, [jnp.full((L,), 1, jnp.int32), idx])
             + plsc.load_gather(rows_v, [jnp.full((L,), 2, jnp.int32), idx]))
        plsc.store_scatter(c_v, [idx], c)
        return 0

    lax.fori_loop(0, MP // L, mid_step, 0)

    def mid_step2(i, _):
        idx = iota + i * L
        i1 = jnp.minimum(idx + 1, N - 1)
        i2 = jnp.minimum(idx + 2, N - 1)
        m = (plsc.load_gather(c_v, [idx])
             + plsc.load_gather(c_v, [i1])
             + plsc.load_gather(c_v, [i2])) * (1.0 / 9.0)
        m = jnp.where(idx < M, m, jnp.inf)
        plsc.store_scatter(midp_v, [idx], m)
        return 0

    lax.fori_loop(0, MP // L, mid_step2, 0)

    xbase = ci * (XA // WPB)
    xs = [plsc.load_gather(ld_v, [iota + (xbase + i * L)]) for i in range(XVPW)]

    def body_a(jc, accs):
        ych = plsc.load_gather(midp_v, [iota + jc * L])
        acc = list(accs)
        for l in range(L):
            yb = _bcast_lane(ych, l)
            for i in range(XVPW):
                d = xs[i] - yb
                acc[i] = jnp.minimum(acc[i], d * d)
        return tuple(acc)

    accs_a = lax.fori_loop(0, MP // L, body_a, (inf_v,) * XVPW)
    s_a = accs_a[0]
    for i in range(1, XVPW):
        s_a = s_a + accs_a[i]

    ybase = ci * (YA // WPB)
    ys = [plsc.load_gather(midp_v, [iota + (ybase + i * L)]) for i in range(YVPW)]

    def body_b(jc, accs):
        xch = plsc.load_gather(ld_v, [iota + jc * L])
        acc = list(accs)
        for l in range(L):
            xb = _bcast_lane(xch, l)
            for i in range(YVPW):
                d = ys[i] - xb
                acc[i] = jnp.minimum(acc[i], d * d)
        return tuple(acc)

    accs_b = lax.fori_loop(0, N // L, body_b, (inf_v,) * YVPW)
    s_b = accs_b[0]
    for i in range(1, YVPW):
        s_b = s_b + accs_b[i]

    out_v[...] = s_a * SCALE_X + s_b * SCALE_Y
    pltpu.sync_copy(out_v, out_hbm.at[wid])


def _chamfer_head(output, lidar):
    mesh = plsc.VectorSubcoreMesh(
        core_axis_name="c", subcore_axis_name="s", num_cores=NC, num_subcores=NS
    )
    f = pl.kernel(
        _chamfer_body,
        out_type=jax.ShapeDtypeStruct((NW, L), jnp.float32),
        mesh=mesh,
        compiler_params=pltpu.CompilerParams(needs_layout_passes=False),
        scratch_types=[
            pltpu.VMEM((3, N), jnp.float32),
            pltpu.VMEM((N,), jnp.float32),
            pltpu.VMEM((MP,), jnp.float32),
            pltpu.VMEM((N,), jnp.float32),
            pltpu.VMEM((L,), jnp.float32),
        ],
    )
    return f(output, lidar)


@jax.jit
def kernel(output, lidar):
    ldr = lidar.reshape(B, N // 128, 128)
    head = _chamfer_head(output, lidar)
    tail = _tail(output, ldr, XA // 128, N // 128, YA // 128, MP // 128)
    return jnp.sum(head) + jnp.sum(tail)


# speedup vs baseline: 1.3918x; 1.0993x over previous
import functools

import jax
import jax.numpy as jnp
from jax import lax
from jax.experimental import pallas as pl
from jax.experimental.pallas import tpu as pltpu
from jax.experimental.pallas import tpu_sc as plsc

B = 4
H = 256
N = 2048
M = 2046
MP = 2048
L = 16
NC = 2
NS = 16
NW = NC * NS
WPB = NW // B

XA = 256
YA = 256
XVPW = XA // WPB // L
YVPW = YA // WPB // L
SCALE_X = 1.0 / (B * N)
SCALE_Y = 1.0 / (B * M)


def _tail_body(rows_ref, ldr_ref, out_ref, *, k_lo, k_hi, j_lo, j_hi):
    r = rows_ref[0, 0]
    c = r[5:6, :] + r[6:7, :] + r[7:8, :]
    c1 = pltpu.roll(c, N - 1, axis=1)
    c2 = pltpu.roll(c, N - 2, axis=1)
    mid = (c + c1 + c2) * (1.0 / 9.0)
    lane = lax.broadcasted_iota(jnp.int32, (1, MP), 1)
    mid = jnp.where(lane < M, mid, jnp.inf)

    ldT = jnp.transpose(ldr_ref[0], (1, 0))
    inf128 = jnp.full((128, 128), jnp.inf, jnp.float32)
    laneid = lax.broadcasted_iota(jnp.int32, (1, 128), 1)

    sx = jnp.float32(0.0)
    for k in range(k_lo, k_hi, 2):
        xc0 = ldT[:, k:k + 1]
        xc1 = ldT[:, k + 1:k + 2]
        acc0 = inf128
        acc1 = inf128
        for j in range(MP // 128):
            yb = mid[:, j * 128:(j + 1) * 128]
            acc0 = jnp.minimum(acc0, (xc0 - yb) ** 2)
            acc1 = jnp.minimum(acc1, (xc1 - yb) ** 2)
        sx = sx + jnp.sum(jnp.min(acc0, axis=1)) + jnp.sum(jnp.min(acc1, axis=1))

    sy = jnp.float32(0.0)
    for j in range(j_lo, j_hi, 2):
        yb0 = mid[:, j * 128:(j + 1) * 128]
        yb1 = mid[:, (j + 1) * 128:(j + 2) * 128]
        acc0 = inf128
        acc1 = inf128
        for k in range(N // 128):
            xc = ldT[:, k:k + 1]
            acc0 = jnp.minimum(acc0, (xc - yb0) ** 2)
            acc1 = jnp.minimum(acc1, (xc - yb1) ** 2)
        v0 = jnp.min(acc0, axis=0, keepdims=True)
        v1 = jnp.min(acc1, axis=0, keepdims=True)
        v0 = jnp.where(laneid + j * 128 < M, v0, 0.0)
        v1 = jnp.where(laneid + (j + 1) * 128 < M, v1, 0.0)
        sy = sy + jnp.sum(v0) + jnp.sum(v1)

    val = sx * SCALE_X + sy * SCALE_Y
    subl = lax.broadcasted_iota(jnp.int32, (8, L), 0)
    lane8 = lax.broadcasted_iota(jnp.int32, (8, L), 1)
    out_ref[...] = jnp.where((subl == 0) & (lane8 == 0), val, 0.0)


SPLIT = 8


def _tail(output, ldr, k_lo, k_hi, j_lo, j_hi):
    body = functools.partial(
        _tail_body, k_lo=k_lo, k_hi=k_hi, j_lo=j_lo, j_hi=j_hi
    )
    return pl.pallas_call(
        body,
        grid=(B,),
        in_specs=[
            pl.BlockSpec((1, 1, 8, N), lambda b: (b, 0, (H // 8) - 1, 0)),
            pl.BlockSpec((1, N // 128, 128), lambda b: (b, 0, 0)),
        ],
        out_specs=pl.BlockSpec((8, L), lambda b: (b, 0)),
        out_shape=jax.ShapeDtypeStruct((B * 8, L), jnp.float32),
    )(output, ldr)


def _iota16():
    return lax.iota(jnp.int32, L)


_GDN = lax.GatherDimensionNumbers(
    offset_dims=(), collapsed_slice_dims=(0,), start_index_map=(0,)
)


def _bcast_lane(v, l):
    idx = jnp.full((L, 1), l, jnp.int32)
    return lax.gather(v, idx, _GDN, (1,),
                      mode=lax.GatherScatterMode.PROMISE_IN_BOUNDS)


def _chamfer_body(out4_hbm, lidar_hbm, out_hbm, rows_v, c_v, midp_v, ld_v, out_v):
    wid = lax.axis_index("c") * NS + lax.axis_index("s")
    b = wid // WPB
    ci = wid % WPB

    pltpu.sync_copy(out4_hbm.at[b, 0, pl.ds(H - 3, 3)], rows_v)
    pltpu.sync_copy(lidar_hbm.at[b, 0], ld_v)

    iota = _iota16()
    inf_v = jnp.full((L,), jnp.inf, jnp.float32)

    def mid_step(i, _):
        idx = iota + i * L
        c = (plsc.load_gather(rows_v, [jnp.full((L,), 0, jnp.int32), idx])
             + plsc.load_gather(rows_v, [jnp.full((L,), 1, jnp.int32), idx])
             + plsc.load_gather(rows_v, [jnp.full((L,), 2, jnp.int32), idx]))
        plsc.store_scatter(c_v, [idx], c)
        return 0

    lax.fori_loop(0, MP // L, mid_step, 0)

    def mid_step2(i, _):
        idx = iota + i * L
        i1 = jnp.minimum(idx + 1, N - 1)
        i2 = jnp.minimum(idx + 2, N - 1)
        m = (plsc.load_gather(c_v, [idx])
             + plsc.load_gather(c_v, [i1])
             + plsc.load_gather(c_v, [i2])) * (1.0 / 9.0)
        m = jnp.where(idx < M, m, jnp.inf)
        plsc.store_scatter(midp_v, [idx], m)
        return 0

    lax.fori_loop(0, MP // L, mid_step2, 0)

    xbase = ci * (XA // WPB)
    xs = [plsc.load_gather(ld_v, [iota + (xbase + i * L)]) for i in range(XVPW)]

    def body_a(jc, accs):
        ych = plsc.load_gather(midp_v, [iota + jc * L])
        acc = list(accs)
        for l in range(L):
            yb = _bcast_lane(ych, l)
            for i in range(XVPW):
                d = xs[i] - yb
                acc[i] = jnp.minimum(acc[i], d * d)
        return tuple(acc)

    accs_a = lax.fori_loop(0, MP // L, body_a, (inf_v,) * XVPW)
    s_a = accs_a[0]
    for i in range(1, XVPW):
        s_a = s_a + accs_a[i]

    ybase = ci * (YA // WPB)
    ys = [plsc.load_gather(midp_v, [iota + (ybase + i * L)]) for i in range(YVPW)]

    def body_b(jc, accs):
        xch = plsc.load_gather(ld_v, [iota + jc * L])
        acc = list(accs)
        for l in range(L):
            xb = _bcast_lane(xch, l)
            for i in range(YVPW):
                d = ys[i] - xb
                acc[i] = jnp.minimum(acc[i], d * d)
        return tuple(acc)

    accs_b = lax.fori_loop(0, N // L, body_b, (inf_v,) * YVPW)
    s_b = accs_b[0]
    for i in range(1, YVPW):
        s_b = s_b + accs_b[i]

    out_v[...] = s_a * SCALE_X + s_b * SCALE_Y
    pltpu.sync_copy(out_v, out_hbm.at[wid])


def _chamfer_head(output, lidar):
    mesh = plsc.VectorSubcoreMesh(
        core_axis_name="c", subcore_axis_name="s", num_cores=NC, num_subcores=NS
    )
    f = pl.kernel(
        _chamfer_body,
        out_type=jax.ShapeDtypeStruct((NW, L), jnp.float32),
        mesh=mesh,
        compiler_params=pltpu.CompilerParams(needs_layout_passes=False),
        scratch_types=[
            pltpu.VMEM((3, N), jnp.float32),
            pltpu.VMEM((N,), jnp.float32),
            pltpu.VMEM((MP,), jnp.float32),
            pltpu.VMEM((N,), jnp.float32),
            pltpu.VMEM((L,), jnp.float32),
        ],
    )
    return f(output, lidar)


@jax.jit
def kernel(output, lidar):
    ldr = lidar.reshape(B, N // 128, 128)
    head = _chamfer_head(output, lidar)
    tail = _tail(output, ldr, XA // 128, N // 128, YA // 128, MP // 128)
    return jnp.sum(head + tail)


# speedup vs baseline: 1.4021x; 1.0074x over previous
import functools

import jax
import jax.numpy as jnp
from jax import lax
from jax.experimental import pallas as pl
from jax.experimental.pallas import tpu as pltpu
from jax.experimental.pallas import tpu_sc as plsc

B = 4
H = 256
N = 2048
M = 2046
MP = 2048
L = 16
NC = 2
NS = 16
NW = NC * NS
WPB = NW // B

XA = 256
YA = 256
XVPW = XA // WPB // L
YVPW = YA // WPB // L
SCALE_X = 1.0 / (B * N)
SCALE_Y = 1.0 / (B * M)


def _tail_body(rows_ref, ldr_ref, out_ref, *, k_lo, k_hi, j_lo, j_hi):
    r = rows_ref[0, 0]
    c = r[5:6, :] + r[6:7, :] + r[7:8, :]
    c1 = pltpu.roll(c, N - 1, axis=1)
    c2 = pltpu.roll(c, N - 2, axis=1)
    mid = (c + c1 + c2) * (1.0 / 9.0)
    lane = lax.broadcasted_iota(jnp.int32, (1, MP), 1)
    mid = jnp.where(lane < M, mid, jnp.inf)

    ldT = jnp.transpose(ldr_ref[0], (1, 0))
    inf128 = jnp.full((128, 128), jnp.inf, jnp.float32)
    laneid = lax.broadcasted_iota(jnp.int32, (1, 128), 1)

    def _groups(lo, hi, g):
        ks = list(range(lo, hi))
        return [ks[i:i + g] for i in range(0, len(ks), g)]

    sx = jnp.float32(0.0)
    for grp in _groups(k_lo, k_hi, 14):
        xcs = [ldT[:, k:k + 1] for k in grp]
        accs = [inf128] * len(grp)
        for j in range(MP // 128):
            yb = mid[:, j * 128:(j + 1) * 128]
            accs = [jnp.minimum(a, (xc - yb) ** 2)
                    for a, xc in zip(accs, xcs)]
        for a in accs:
            sx = sx + jnp.sum(jnp.min(a, axis=1))

    sy = jnp.float32(0.0)
    for grp in _groups(j_lo, j_hi, 14):
        ybs = [mid[:, j * 128:(j + 1) * 128] for j in grp]
        accs = [inf128] * len(grp)
        for k in range(N // 128):
            xc = ldT[:, k:k + 1]
            accs = [jnp.minimum(a, (xc - yb) ** 2)
                    for a, yb in zip(accs, ybs)]
        for a, j in zip(accs, grp):
            v = jnp.min(a, axis=0, keepdims=True)
            v = jnp.where(laneid + j * 128 < M, v, 0.0)
            sy = sy + jnp.sum(v)

    val = sx * SCALE_X + sy * SCALE_Y
    subl = lax.broadcasted_iota(jnp.int32, (8, L), 0)
    lane8 = lax.broadcasted_iota(jnp.int32, (8, L), 1)
    out_ref[...] = jnp.where((subl == 0) & (lane8 == 0), val, 0.0)


SPLIT = 8


def _tail(output, ldr, k_lo, k_hi, j_lo, j_hi):
    body = functools.partial(
        _tail_body, k_lo=k_lo, k_hi=k_hi, j_lo=j_lo, j_hi=j_hi
    )
    return pl.pallas_call(
        body,
        grid=(B,),
        in_specs=[
            pl.BlockSpec((1, 1, 8, N), lambda b: (b, 0, (H // 8) - 1, 0)),
            pl.BlockSpec((1, N // 128, 128), lambda b: (b, 0, 0)),
        ],
        out_specs=pl.BlockSpec((8, L), lambda b: (b, 0)),
        out_shape=jax.ShapeDtypeStruct((B * 8, L), jnp.float32),
    )(output, ldr)


def _iota16():
    return lax.iota(jnp.int32, L)


_GDN = lax.GatherDimensionNumbers(
    offset_dims=(), collapsed_slice_dims=(0,), start_index_map=(0,)
)


def _bcast_lane(v, l):
    idx = jnp.full((L, 1), l, jnp.int32)
    return lax.gather(v, idx, _GDN, (1,),
                      mode=lax.GatherScatterMode.PROMISE_IN_BOUNDS)


def _chamfer_body(out4_hbm, lidar_hbm, out_hbm, rows_v, c_v, midp_v, ld_v, out_v):
    wid = lax.axis_index("c") * NS + lax.axis_index("s")
    b = wid // WPB
    ci = wid % WPB

    pltpu.sync_copy(out4_hbm.at[b, 0, pl.ds(H - 3, 3)], rows_v)
    pltpu.sync_copy(lidar_hbm.at[b, 0], ld_v)

    iota = _iota16()
    inf_v = jnp.full((L,), jnp.inf, jnp.float32)

    def mid_step(i, _):
        idx = iota + i * L
        c = (plsc.load_gather(rows_v, [jnp.full((L,), 0, jnp.int32), idx])
             + plsc.load_gather(rows_v, [jnp.full((L,), 1, jnp.int32), idx])
             + plsc.load_gather(rows_v, [jnp.full((L,), 2, jnp.int32), idx]))
        plsc.store_scatter(c_v, [idx], c)
        return 0

    lax.fori_loop(0, MP // L, mid_step, 0)

    def mid_step2(i, _):
        idx = iota + i * L
        i1 = jnp.minimum(idx + 1, N - 1)
        i2 = jnp.minimum(idx + 2, N - 1)
        m = (plsc.load_gather(c_v, [idx])
             + plsc.load_gather(c_v, [i1])
             + plsc.load_gather(c_v, [i2])) * (1.0 / 9.0)
        m = jnp.where(idx < M, m, jnp.inf)
        plsc.store_scatter(midp_v, [idx], m)
        return 0

    lax.fori_loop(0, MP // L, mid_step2, 0)

    xbase = ci * (XA // WPB)
    xs = [plsc.load_gather(ld_v, [iota + (xbase + i * L)]) for i in range(XVPW)]

    def body_a(jc, accs):
        ych = plsc.load_gather(midp_v, [iota + jc * L])
        acc = list(accs)
        for l in range(L):
            yb = _bcast_lane(ych, l)
            for i in range(XVPW):
                d = xs[i] - yb
                acc[i] = jnp.minimum(acc[i], d * d)
        return tuple(acc)

    accs_a = lax.fori_loop(0, MP // L, body_a, (inf_v,) * XVPW)
    s_a = accs_a[0]
    for i in range(1, XVPW):
        s_a = s_a + accs_a[i]

    ybase = ci * (YA // WPB)
    ys = [plsc.load_gather(midp_v, [iota + (ybase + i * L)]) for i in range(YVPW)]

    def body_b(jc, accs):
        xch = plsc.load_gather(ld_v, [iota + jc * L])
        acc = list(accs)
        for l in range(L):
            xb = _bcast_lane(xch, l)
            for i in range(YVPW):
                d = ys[i] - xb
                acc[i] = jnp.minimum(acc[i], d * d)
        return tuple(acc)

    accs_b = lax.fori_loop(0, N // L, body_b, (inf_v,) * YVPW)
    s_b = accs_b[0]
    for i in range(1, YVPW):
        s_b = s_b + accs_b[i]

    out_v[...] = s_a * SCALE_X + s_b * SCALE_Y
    pltpu.sync_copy(out_v, out_hbm.at[wid])


def _chamfer_head(output, lidar):
    mesh = plsc.VectorSubcoreMesh(
        core_axis_name="c", subcore_axis_name="s", num_cores=NC, num_subcores=NS
    )
    f = pl.kernel(
        _chamfer_body,
        out_type=jax.ShapeDtypeStruct((NW, L), jnp.float32),
        mesh=mesh,
        compiler_params=pltpu.CompilerParams(needs_layout_passes=False),
        scratch_types=[
            pltpu.VMEM((3, N), jnp.float32),
            pltpu.VMEM((N,), jnp.float32),
            pltpu.VMEM((MP,), jnp.float32),
            pltpu.VMEM((N,), jnp.float32),
            pltpu.VMEM((L,), jnp.float32),
        ],
    )
    return f(output, lidar)


@jax.jit
def kernel(output, lidar):
    ldr = lidar.reshape(B, N // 128, 128)
    head = _chamfer_head(output, lidar)
    tail = _tail(output, ldr, XA // 128, N // 128, YA // 128, MP // 128)
    return jnp.sum(head + tail)
